# baseline (speedup 1.0000x reference)
import functools

import jax
import jax.numpy as jnp
from jax import lax
from jax.experimental import pallas as pl
from jax.experimental.pallas import tpu as pltpu
from jax.experimental.pallas import tpu_sc as plsc

_ACT_N = 128
_L = 16
_CHUNKS = _ACT_N // _L


def _qagent_body(q_hbm, obs_hbm, out_hbm, idx_v, row_v, res_v, sem):
    cid = lax.axis_index("c")
    sid = lax.axis_index("s")

    @pl.when(jnp.logical_and(cid == 0, sid == 0))
    def _():
        pltpu.sync_copy(obs_hbm, idx_v)
        pltpu.async_copy(q_hbm.at[idx_v], row_v, sem).wait()

        iota = lax.iota(jnp.int32, _L)
        perms = [jnp.bitwise_xor(iota, 1 << k) for k in range(4)]

        dnums = lax.GatherDimensionNumbers(
            offset_dims=(), collapsed_slice_dims=(0,), start_index_map=(0,))

        def _shuffle(v, p):
            return lax.gather(
                v, p[:, None], dimension_numbers=dnums, slice_sizes=(1,),
                mode=lax.GatherScatterMode.PROMISE_IN_BOUNDS)

        m = row_v[0, pl.ds(0, _L)]
        for j in range(1, _CHUNKS):
            m = jnp.maximum(m, row_v[0, pl.ds(j * _L, _L)])
        for p in perms:
            m = jnp.maximum(m, _shuffle(m, p))

        best = jnp.full((_L,), _ACT_N, jnp.int32)
        for j in range(_CHUNKS):
            c = row_v[0, pl.ds(j * _L, _L)]
            cand = jnp.where(c == m, iota + j * _L, _ACT_N)
            best = jnp.minimum(best, cand)
        for p in perms:
            best = jnp.minimum(best, _shuffle(best, p))

        res_v[...] = best
        pltpu.sync_copy(res_v, out_hbm)


@functools.partial(
    pl.kernel,
    mesh=plsc.VectorSubcoreMesh(core_axis_name="c", subcore_axis_name="s"),
    out_type=jax.ShapeDtypeStruct((_L,), jnp.int32),
    scratch_types=[
        pltpu.VMEM((1,), jnp.int32),
        pltpu.VMEM((1, _ACT_N), jnp.float32),
        pltpu.VMEM((_L,), jnp.int32),
        pltpu.SemaphoreType.DMA,
    ],
)
def _qagent_sc(q_hbm, obs_hbm, out_hbm, idx_v, row_v, res_v, sem):
    _qagent_body(q_hbm, obs_hbm, out_hbm, idx_v, row_v, res_v, sem)


def kernel(Q, obs):
    obs_v = jnp.reshape(jnp.asarray(obs, jnp.int32), (1,))
    out = _qagent_sc(Q, obs_v)
    return out[0]


# speedup vs baseline: 1.0765x; 1.0765x over previous
import functools

import jax
import jax.numpy as jnp
from jax import lax
from jax.experimental import pallas as pl
from jax.experimental.pallas import tpu as pltpu
from jax.experimental.pallas import tpu_sc as plsc

_ACT_N = 128
_L = 16
_CHUNKS = _ACT_N // _L


def _qagent_body(q_hbm, obs_hbm, out_hbm, idx_v, row_v, res_v, sem):
    cid = lax.axis_index("c")
    sid = lax.axis_index("s")

    @pl.when(jnp.logical_and(cid == 0, sid == 0))
    def _():
        pltpu.sync_copy(obs_hbm, idx_v)
        pltpu.async_copy(q_hbm.at[idx_v], row_v, sem).wait()

        iota = lax.iota(jnp.int32, _L)
        perms = [jnp.bitwise_xor(iota, 1 << k) for k in range(4)]

        dnums = lax.GatherDimensionNumbers(
            offset_dims=(), collapsed_slice_dims=(0,), start_index_map=(0,))

        def _shuffle(v, p):
            return lax.gather(
                v, p[:, None], dimension_numbers=dnums, slice_sizes=(1,),
                mode=lax.GatherScatterMode.PROMISE_IN_BOUNDS)

        m = row_v[0, pl.ds(0, _L)]
        for j in range(1, _CHUNKS):
            m = jnp.maximum(m, row_v[0, pl.ds(j * _L, _L)])
        for p in perms:
            m = jnp.maximum(m, _shuffle(m, p))

        best = jnp.full((_L,), _ACT_N, jnp.int32)
        for j in range(_CHUNKS):
            c = row_v[0, pl.ds(j * _L, _L)]
            cand = jnp.where(c == m, iota + j * _L, _ACT_N)
            best = jnp.minimum(best, cand)
        for p in perms:
            best = jnp.minimum(best, _shuffle(best, p))

        res_v[...] = best
        pltpu.sync_copy(res_v, out_hbm)


@functools.partial(
    pl.kernel,
    mesh=plsc.VectorSubcoreMesh(
        core_axis_name="c", subcore_axis_name="s", num_cores=1),
    out_type=jax.ShapeDtypeStruct((_L,), jnp.int32),
    scratch_types=[
        pltpu.VMEM((1,), jnp.int32),
        pltpu.VMEM((1, _ACT_N), jnp.float32),
        pltpu.VMEM((_L,), jnp.int32),
        pltpu.SemaphoreType.DMA,
    ],
)
def _qagent_sc(q_hbm, obs_hbm, out_hbm, idx_v, row_v, res_v, sem):
    _qagent_body(q_hbm, obs_hbm, out_hbm, idx_v, row_v, res_v, sem)


def kernel(Q, obs):
    obs_v = jnp.reshape(jnp.asarray(obs, jnp.int32), (1,))
    out = _qagent_sc(Q, obs_v)
    return out[0]
